# baseline (speedup 1.0000x reference)
import functools

import jax
import jax.numpy as jnp
from jax import lax
from jax.experimental import pallas as pl
from jax.experimental.pallas import tpu as pltpu
from jax.experimental.pallas import tpu_sc as plsc

B = 16384
D = 32
V = 100001
VPAD = 100096
NUM_CORES = 2
NUM_SUBCORES = 16
NW = NUM_CORES * NUM_SUBCORES
N_TILES = VPAD // 128
SLAB_TILES = 25
SLAB_COLS = SLAB_TILES * 128
N_BANDS = D // 8
POS_SHIFT = 14
N_STRIPES = 8
STRIPE = B // N_STRIPES
STRIPE_CAP = STRIPE + 16


def _make_kernel():
    mesh = plsc.VectorSubcoreMesh(core_axis_name="c", subcore_axis_name="s")

    @functools.partial(
        pl.kernel,
        mesh=mesh,
        out_type=jax.ShapeDtypeStruct((B * D,), jnp.float32),
        scratch_types=[
            pltpu.VMEM((2 * 2048,), jnp.int32),
            pltpu.VMEM((N_BANDS, 8, SLAB_COLS), jnp.float32),
            pltpu.VMEM((N_STRIPES * STRIPE_CAP,), jnp.int32),
            pltpu.VMEM((16, D), jnp.float32),
            pltpu.SemaphoreType.DMA,
            pltpu.SemaphoreType.DMA,
            pltpu.SemaphoreType.DMA,
        ],
        compiler_params=pltpu.CompilerParams(
            use_tc_tiling_on_sc=True, needs_layout_passes=False
        ),
    )
    def k(
        tbl_hbm, idx_hbm, out_hbm, idx_v, slab_v, hits_v, ring_v,
        sem, osem, isem,
    ):
        wid = lax.axis_index("s") * NUM_CORES + lax.axis_index("c")
        small = jnp.int32(25 * 14)
        t0 = jnp.where(wid < 14, 25 * wid, small + 24 * (wid - 14))
        ntc = jnp.where(wid < 14, 25, 24)
        slab_t0 = jnp.minimum(t0, N_TILES - SLAB_TILES)
        slab_c0 = slab_t0 * 128
        sel_a = t0 * 128
        sel_b = (t0 + ntc) * 128

        slab_cps = []
        for band in range(N_BANDS):
            slab_cps.append(
                pltpu.async_copy(
                    tbl_hbm.at[pl.ds(band * 8, 8), pl.ds(slab_c0, SLAB_COLS)],
                    slab_v.at[band],
                    sem,
                )
            )
        lane = lax.iota(jnp.int32, 16)
        zeros = jnp.full((16,), 0, jnp.int32)
        sel_a_v = zeros + sel_a
        sel_b_v = zeros + sel_b
        shifted_c0 = zeros + lax.shift_left(slab_c0, POS_SHIFT)

        N_BLOCKS = B // 2048
        idx_cp = pltpu.async_copy(
            idx_hbm.at[pl.ds(0, 2048)], idx_v.at[pl.ds(0, 2048)], isem
        )
        offs = (jnp.int32(0),) * N_STRIPES
        for blk in range(N_BLOCKS):
            idx_cp.wait()
            if blk + 1 < N_BLOCKS:
                idx_cp = pltpu.async_copy(
                    idx_hbm.at[pl.ds((blk + 1) * 2048, 2048)],
                    idx_v.at[pl.ds(((blk + 1) % 2) * 2048, 2048)],
                    isem,
                )

            def sel_body(j, offs, blk=blk):
                new_offs = []
                for s in range(N_STRIPES):
                    local0 = (j * N_STRIPES + s) * 16
                    vec = idx_v[pl.ds((blk % 2) * 2048 + local0, 16)]
                    m = jnp.logical_and(vec >= sel_a_v, vec < sel_b_v)
                    cnt = plsc.all_reduce_population_count(m)
                    packed = (
                        (lane + (blk * 2048 + local0))
                        + lax.shift_left(vec, POS_SHIFT)
                        - shifted_c0
                    )
                    plsc.store_compressed(
                        hits_v.at[pl.ds(s * STRIPE_CAP + offs[s], 16)],
                        packed,
                        mask=m,
                    )
                    new_offs.append(offs[s] + cnt[0])
                return tuple(new_offs)

            offs = lax.fori_loop(0, 2048 // (16 * N_STRIPES), sel_body, offs)

        for s in range(N_STRIPES):
            first_vec = hits_v[pl.ds(s * STRIPE_CAP, 16)]

            @pl.when(offs[s] > 0)
            def _(s=s, first_vec=first_vec):
                hits_v[pl.ds(s * STRIPE_CAP + offs[s], 16)] = (
                    zeros + first_vec[0]
                )

        for c in slab_cps:
            c.wait()

        band_idx, sub_idx = [], []
        for h in range(2):
            d = lane + h * 16
            band_idx.append(lax.shift_right_logical(d, 3))
            sub_idx.append(d & 7)
        pos_mask = zeros + ((1 << POS_SHIFT) - 1)

        for s in range(N_STRIPES):
            n_grp = lax.shift_right_logical(offs[s] + 15, 4)

            def ext_body(eg, carry, s=s):
                pk = hits_v[pl.ds(s * STRIPE_CAP + eg * 16, 16)]
                pos_v = pk & pos_mask
                col_v = lax.shift_right_logical(pk, POS_SHIFT)
                cps = []
                for e in range(16):
                    col = zeros + col_v[e]
                    for h in range(2):
                        ring_v[e, pl.ds(h * 16, 16)] = plsc.load_gather(
                            slab_v, [band_idx[h], sub_idx[h], col]
                        )
                    cps.append(
                        pltpu.async_copy(
                            ring_v.at[e],
                            out_hbm.at[pl.ds(pos_v[e] * D, D)],
                            osem,
                        )
                    )
                for c in cps:
                    c.wait()
                return carry

            lax.fori_loop(0, n_grp, ext_body, jnp.int32(0))

    return k


@jax.jit
def kernel(template_table, eventids):
    idx = eventids.astype(jnp.int32)
    tbl_t = template_table.T
    out1d = _make_kernel()(tbl_t, idx)
    return out1d.reshape(B, D)


# speedup vs baseline: 1.0120x; 1.0120x over previous
import functools

import jax
import jax.numpy as jnp
from jax import lax
from jax.experimental import pallas as pl
from jax.experimental.pallas import tpu as pltpu
from jax.experimental.pallas import tpu_sc as plsc

B = 16384
D = 32
V = 100001
VPAD = 100096
NUM_CORES = 2
NUM_SUBCORES = 16
NW = NUM_CORES * NUM_SUBCORES
N_TILES = VPAD // 128
SLAB_TILES = 25
SLAB_COLS = SLAB_TILES * 128
N_BANDS = D // 8
POS_SHIFT = 14
N_STRIPES = 8
STRIPE = B // N_STRIPES
STRIPE_CAP = STRIPE + 16


def _make_kernel():
    mesh = plsc.VectorSubcoreMesh(core_axis_name="c", subcore_axis_name="s")

    @functools.partial(
        pl.kernel,
        mesh=mesh,
        out_type=jax.ShapeDtypeStruct((B * D,), jnp.float32),
        scratch_types=[
            pltpu.VMEM((2 * 2048,), jnp.int32),
            pltpu.VMEM((N_BANDS, 8, SLAB_COLS), jnp.float32),
            pltpu.VMEM((N_STRIPES * STRIPE_CAP,), jnp.int32),
            pltpu.VMEM((16, D), jnp.float32),
            pltpu.SemaphoreType.DMA,
            pltpu.SemaphoreType.DMA,
            pltpu.SemaphoreType.DMA,
        ],
        compiler_params=pltpu.CompilerParams(
            use_tc_tiling_on_sc=True, needs_layout_passes=False
        ),
    )
    def k(
        tbl_hbm, idx_hbm, out_hbm, idx_v, slab_v, hits_v, ring_v,
        sem, osem, isem,
    ):
        wid = lax.axis_index("s") * NUM_CORES + lax.axis_index("c")
        small = jnp.int32(25 * 14)
        t0 = jnp.where(wid < 14, 25 * wid, small + 24 * (wid - 14))
        ntc = jnp.where(wid < 14, 25, 24)
        slab_t0 = jnp.minimum(t0, N_TILES - SLAB_TILES)
        slab_c0 = slab_t0 * 128
        sel_a = t0 * 128
        sel_b = (t0 + ntc) * 128

        slab_cps = []
        for band in range(N_BANDS):
            slab_cps.append(
                pltpu.async_copy(
                    tbl_hbm.at[pl.ds(band * 8, 8), pl.ds(slab_c0, SLAB_COLS)],
                    slab_v.at[band],
                    sem,
                )
            )
        lane = lax.iota(jnp.int32, 16)
        zeros = jnp.full((16,), 0, jnp.int32)
        sel_a_v = zeros + sel_a
        sel_b_v = zeros + sel_b
        shifted_c0 = zeros + lax.shift_left(slab_c0, POS_SHIFT)

        N_BLOCKS = B // 2048
        idx_cp = pltpu.async_copy(
            idx_hbm.at[pl.ds(0, 2048)], idx_v.at[pl.ds(0, 2048)], isem
        )
        offs = (jnp.int32(0),) * N_STRIPES
        for blk in range(N_BLOCKS):
            idx_cp.wait()
            if blk + 1 < N_BLOCKS:
                idx_cp = pltpu.async_copy(
                    idx_hbm.at[pl.ds((blk + 1) * 2048, 2048)],
                    idx_v.at[pl.ds(((blk + 1) % 2) * 2048, 2048)],
                    isem,
                )

            def sel_body(j, offs, blk=blk):
                new_offs = []
                for s in range(N_STRIPES):
                    local0 = (j * N_STRIPES + s) * 16
                    vec = idx_v[pl.ds((blk % 2) * 2048 + local0, 16)]
                    m = jnp.logical_and(vec >= sel_a_v, vec < sel_b_v)
                    cnt = plsc.all_reduce_population_count(m)
                    packed = (
                        (lane + (blk * 2048 + local0))
                        + lax.shift_left(vec, POS_SHIFT)
                        - shifted_c0
                    )
                    plsc.store_compressed(
                        hits_v.at[pl.ds(s * STRIPE_CAP + offs[s], 16)],
                        packed,
                        mask=m,
                    )
                    new_offs.append(offs[s] + cnt[0])
                return tuple(new_offs)

            offs = lax.fori_loop(0, 2048 // (16 * N_STRIPES), sel_body, offs)

        for s in range(N_STRIPES):
            first_vec = hits_v[pl.ds(s * STRIPE_CAP, 16)]

            @pl.when(offs[s] > 0)
            def _(s=s, first_vec=first_vec):
                hits_v[pl.ds(s * STRIPE_CAP + offs[s], 16)] = (
                    zeros + first_vec[0]
                )

        for c in slab_cps:
            c.wait()

        pos_mask = zeros + ((1 << POS_SHIFT) - 1)
        row_lane = lane

        for s in range(N_STRIPES):
            n_grp = lax.shift_right_logical(offs[s] + 15, 4)

            def ext_body(eg, carry, s=s):
                pk = hits_v[pl.ds(s * STRIPE_CAP + eg * 16, 16)]
                pos_v = pk & pos_mask
                col_v = lax.shift_right_logical(pk, POS_SHIFT)
                for d in range(D):
                    vals = plsc.load_gather(
                        slab_v,
                        [zeros + (d // 8), zeros + (d % 8), col_v],
                    )
                    plsc.store_scatter(
                        ring_v, [row_lane, zeros + d], vals
                    )
                cps = []
                for e in range(16):
                    cps.append(
                        pltpu.async_copy(
                            ring_v.at[e],
                            out_hbm.at[pl.ds(pos_v[e] * D, D)],
                            osem,
                        )
                    )
                for c in cps:
                    c.wait()
                return carry

            lax.fori_loop(0, n_grp, ext_body, jnp.int32(0))

    return k


@jax.jit
def kernel(template_table, eventids):
    idx = eventids.astype(jnp.int32)
    tbl_t = template_table.T
    out1d = _make_kernel()(tbl_t, idx)
    return out1d.reshape(B, D)
